# baseline (speedup 1.0000x reference)
import functools

import jax
import jax.numpy as jnp
from jax import lax
from jax.experimental import pallas as pl
from jax.experimental.pallas import tpu as pltpu
from jax.experimental.pallas import tpu_sc as plsc

_NC = 2
_NS = 16
_NW = _NC * _NS
_LANES = 16


def _row_lse_body(t_ref, lse_ref):
    t = t_ref[...]
    m = jnp.max(t, axis=1, keepdims=True)
    s = jnp.sum(jnp.exp(t - m), axis=1, keepdims=True)
    lse_ref[...] = m + jnp.log(s)


def _loss_body(n_tokens, p_ref, out_ref):
    out_ref[...] = jnp.sum(p_ref[...]).reshape(1, 1) * (1.0 / n_tokens)


def _make_sc_loss(vocab, dim, n_tokens, lchunk):
    per_w = n_tokens // _NW
    n_lchunks = per_w // lchunk
    mesh = plsc.VectorSubcoreMesh(core_axis_name="c", subcore_axis_name="s")

    @functools.partial(
        pl.kernel,
        mesh=mesh,
        compiler_params=pltpu.CompilerParams(use_tc_tiling_on_sc=False),
        out_type=jax.ShapeDtypeStruct((_NW, _LANES), jnp.float32),
        scratch_types=[
            pltpu.VMEM((per_w,), jnp.int32),
            pltpu.VMEM((per_w,), jnp.int32),
            pltpu.VMEM((per_w,), jnp.int32),
            pltpu.VMEM((per_w,), jnp.float32),
            pltpu.VMEM((per_w,), jnp.float32),
            pltpu.VMEM((_LANES,), jnp.float32),
            pltpu.SemaphoreType.DMA,
            pltpu.SemaphoreType.DMA,
        ],
    )
    def sc_loss(tflat_hbm, x_hbm, y_hbm, lse_hbm, part_hbm,
                xv, yv, fbuf, pbuf, lbuf, accv, psem, lsem):
        wid = lax.axis_index("s") * _NC + lax.axis_index("c")
        base = wid * per_w
        pltpu.sync_copy(x_hbm.at[pl.ds(base, per_w)], xv)
        pltpu.sync_copy(y_hbm.at[pl.ds(base, per_w)], yv)

        def issue(m, _):
            for j in range(lchunk // _LANES):
                off = m * lchunk + j * _LANES
                xvec = xv[pl.ds(off, _LANES)]
                yvec = yv[pl.ds(off, _LANES)]
                fbuf[pl.ds(off, _LANES)] = xvec * dim + yvec
            sl = pl.ds(m * lchunk, lchunk)
            pltpu.async_copy(tflat_hbm.at[fbuf.at[sl]], pbuf.at[sl], psem)
            pltpu.async_copy(lse_hbm.at[xv.at[sl]], lbuf.at[sl], lsem)
            return 0
        lax.fori_loop(0, n_lchunks, issue, 0)

        pltpu.make_async_copy(tflat_hbm.at[pl.ds(0, per_w)], pbuf, psem).wait()
        pltpu.make_async_copy(tflat_hbm.at[pl.ds(0, per_w)], lbuf, lsem).wait()

        def acc_body(m, acc):
            off = m * _LANES
            return acc + (lbuf[pl.ds(off, _LANES)] - pbuf[pl.ds(off, _LANES)])

        acc = lax.fori_loop(0, per_w // _LANES, acc_body,
                            jnp.zeros((_LANES,), jnp.float32))
        accv[...] = acc
        pltpu.sync_copy(accv, part_hbm.at[wid])

    return sc_loss


def _mm_body(vocab, batch, tb, xt_ref, tblt_ref, out_ref):
    iota_v = lax.broadcasted_iota(jnp.int32, (vocab, tb * batch), 0)
    xcols = xt_ref[...].reshape(1, tb * batch)
    oh = (iota_v == xcols).astype(jnp.bfloat16)
    res = lax.dot_general(
        tblt_ref[...], oh, (((1,), (0,)), ((), ())),
        preferred_element_type=jnp.float32)
    for k in range(tb):
        out_ref[k] = res[:, k * batch:(k + 1) * batch]


def kernel(X, y, embedding_table):
    B, T = X.shape
    vocab, dim = embedding_table.shape
    n_tokens = B * T
    xf = X.reshape(-1)
    yf = y.reshape(-1)

    row_lse = pl.pallas_call(
        _row_lse_body,
        out_shape=jax.ShapeDtypeStruct((vocab, 1), jnp.float32),
    )(embedding_table)

    sc_loss = _make_sc_loss(vocab, dim, n_tokens, lchunk=128)
    tflat = jnp.pad(embedding_table.reshape(-1), (0, _LANES))
    partials = sc_loss(tflat, xf, yf, row_lse.reshape(-1))

    loss2d = pl.pallas_call(
        functools.partial(_loss_body, n_tokens),
        out_shape=jax.ShapeDtypeStruct((1, 1), jnp.float32),
    )(partials)

    tblt = embedding_table.astype(jnp.bfloat16).T
    tb = 2
    xt = X.T.reshape(T, 1, B)
    out_phys = pl.pallas_call(
        functools.partial(_mm_body, vocab, B, tb),
        grid=(T // tb,),
        in_specs=[
            pl.BlockSpec((tb, 1, B), lambda t: (t, 0, 0)),
            pl.BlockSpec((dim, vocab), lambda t: (0, 0)),
        ],
        out_specs=pl.BlockSpec((tb, dim, B), lambda t: (t, 0, 0)),
        out_shape=jax.ShapeDtypeStruct((T, dim, B), jnp.float32),
    )(xt, tblt)
    logits = jnp.transpose(out_phys, (2, 0, 1))

    return logits, loss2d[0, 0]
